# baseline (speedup 1.0000x reference)
import jax
import jax.numpy as jnp
from jax import lax
from jax.experimental import pallas as pl
from jax.experimental.pallas import tpu as pltpu
from jax.experimental.pallas import tpu_sc as plsc

_ROWS = 100000
_COLS = 256
_TOTAL = _ROWS * _COLS
_NC, _NS = 2, 16
_NW = _NC * _NS
_PER_W = _TOTAL // _NW
_CHUNK = 50_000
_NCH = _PER_W // _CHUNK


def _sc_copy_body(src, dst, b0, b1, si0, si1, so0, so1):
    wid = lax.axis_index("c") * _NS + lax.axis_index("s")
    base = wid * _PER_W
    bufs = (b0, b1)
    in_sems = (si0, si1)
    out_sems = (so0, so1)
    out_pending = [None, None]
    for k in range(_NCH):
        i = k % 2
        off = base + k * _CHUNK
        if out_pending[i] is not None:
            out_pending[i].wait()
        pltpu.async_copy(src.at[pl.ds(off, _CHUNK)], bufs[i], in_sems[i]).wait()
        out_pending[i] = pltpu.async_copy(
            bufs[i], dst.at[pl.ds(off, _CHUNK)], out_sems[i])
    out_pending[0].wait()
    out_pending[1].wait()


def kernel(weight):
    flat = weight.reshape(_TOTAL)
    mesh = plsc.VectorSubcoreMesh(core_axis_name="c", subcore_axis_name="s")
    out = pl.kernel(
        _sc_copy_body,
        out_type=jax.ShapeDtypeStruct((_TOTAL,), jnp.float32),
        mesh=mesh,
        scratch_types=[
            pltpu.VMEM((_CHUNK,), jnp.float32),
            pltpu.VMEM((_CHUNK,), jnp.float32),
            pltpu.SemaphoreType.DMA,
            pltpu.SemaphoreType.DMA,
            pltpu.SemaphoreType.DMA,
            pltpu.SemaphoreType.DMA,
        ],
    )(flat)
    return out.reshape(_ROWS, _COLS)


# speedup vs baseline: 3.0210x; 3.0210x over previous
import jax
import jax.numpy as jnp
from jax import lax
from jax.experimental import pallas as pl
from jax.experimental.pallas import tpu as pltpu
from jax.experimental.pallas import tpu_sc as plsc

_ROWS = 100000
_COLS = 256
_NC, _NS = 2, 16
_NW = _NC * _NS
_CHUNK_ROWS = 200
_NCHUNKS = _ROWS // _CHUNK_ROWS
_FULL_STEPS = _NCHUNKS // _NW
_TAIL = _NCHUNKS - _FULL_STEPS * _NW


def _sc_copy_body(src, dst, b0, b1, si0, si1, so0, so1):
    wid = lax.axis_index("c") * _NS + lax.axis_index("s")
    bufs = (b0, b1)
    in_sems = (si0, si1)
    out_sems = (so0, so1)

    def row0(k):
        return (wid + k * _NW) * _CHUNK_ROWS

    def start_in(k):
        i = k % 2
        return pltpu.async_copy(
            src.at[pl.ds(row0(k), _CHUNK_ROWS)], bufs[i], in_sems[i])

    def start_out(k):
        i = k % 2
        return pltpu.async_copy(
            bufs[i], dst.at[pl.ds(row0(k), _CHUNK_ROWS)], out_sems[i])

    in_p = [start_in(0), start_in(1)]
    out_p = [None, None]
    for k in range(_FULL_STEPS):
        i = k % 2
        in_p[i].wait()
        out_p[i] = start_out(k)
        nk = k + 2
        if nk < _FULL_STEPS:
            out_p[i].wait()
            in_p[i] = start_in(nk)
    for oc in out_p:
        if oc is not None:
            oc.wait()

    @pl.when(wid < _TAIL)
    def _():
        r0 = (_FULL_STEPS * _NW + wid) * _CHUNK_ROWS
        pltpu.sync_copy(src.at[pl.ds(r0, _CHUNK_ROWS)], b0)
        pltpu.sync_copy(b0, dst.at[pl.ds(r0, _CHUNK_ROWS)])


def kernel(weight):
    mesh = plsc.VectorSubcoreMesh(core_axis_name="c", subcore_axis_name="s")
    return pl.kernel(
        _sc_copy_body,
        out_type=jax.ShapeDtypeStruct((_ROWS, _COLS), jnp.float32),
        mesh=mesh,
        scratch_types=[
            pltpu.VMEM((_CHUNK_ROWS, _COLS), jnp.float32),
            pltpu.VMEM((_CHUNK_ROWS, _COLS), jnp.float32),
            pltpu.SemaphoreType.DMA,
            pltpu.SemaphoreType.DMA,
            pltpu.SemaphoreType.DMA,
            pltpu.SemaphoreType.DMA,
        ],
    )(weight)


# speedup vs baseline: 3.0848x; 1.0211x over previous
import jax
import jax.numpy as jnp
from jax import lax
from jax.experimental import pallas as pl
from jax.experimental.pallas import tpu as pltpu
from jax.experimental.pallas import tpu_sc as plsc

_ROWS = 100000
_COLS = 256
_NC, _NS = 2, 16
_NW = _NC * _NS
_CHUNK_ROWS = 240
_STEPS = 13
_TAIL_BASE = _NW * _STEPS * _CHUNK_ROWS
_TAIL_ROWS = _ROWS - _TAIL_BASE


def _sc_copy_body(src, dst, b0, b1, si0, si1, so0, so1):
    cid = lax.axis_index("c")
    wid = cid * _NS + lax.axis_index("s")
    bufs = (b0, b1)
    in_sems = (si0, si1)
    out_sems = (so0, so1)

    def row0(k):
        return (wid + k * _NW) * _CHUNK_ROWS

    def start_in(k):
        i = k % 2
        return pltpu.async_copy(
            src.at[pl.ds(row0(k), _CHUNK_ROWS)], bufs[i], in_sems[i])

    def start_out(k):
        i = k % 2
        return pltpu.async_copy(
            bufs[i], dst.at[pl.ds(row0(k), _CHUNK_ROWS)], out_sems[i])

    in_p = [start_in(0), start_in(1)]
    out_p = [None, None]
    for k in range(_STEPS):
        i = k % 2
        in_p[i].wait()
        out_p[i] = start_out(k)
        nk = k + 2
        if nk < _STEPS:
            out_p[i].wait()
            in_p[i] = start_in(nk)
    for oc in out_p:
        if oc is not None:
            oc.wait()

    half = _TAIL_ROWS // _NC

    @pl.when(lax.axis_index("s") == _NS - 1)
    def _():
        r0 = _TAIL_BASE + cid * half
        tb = b0.at[pl.ds(0, half)]
        pltpu.sync_copy(src.at[pl.ds(r0, half)], tb)
        pltpu.sync_copy(tb, dst.at[pl.ds(r0, half)])


def kernel(weight):
    mesh = plsc.VectorSubcoreMesh(core_axis_name="c", subcore_axis_name="s")
    return pl.kernel(
        _sc_copy_body,
        out_type=jax.ShapeDtypeStruct((_ROWS, _COLS), jnp.float32),
        mesh=mesh,
        scratch_types=[
            pltpu.VMEM((_CHUNK_ROWS, _COLS), jnp.float32),
            pltpu.VMEM((_CHUNK_ROWS, _COLS), jnp.float32),
            pltpu.SemaphoreType.DMA,
            pltpu.SemaphoreType.DMA,
            pltpu.SemaphoreType.DMA,
            pltpu.SemaphoreType.DMA,
        ],
    )(weight)
